# baseline (speedup 1.0000x reference)
import functools

import jax
import jax.numpy as jnp
from jax import lax
from jax.experimental import pallas as pl
from jax.experimental.pallas import tpu as pltpu
from jax.experimental.pallas import tpu_sc as plsc

_NUM_BOXES = 100000
_ROW = 128
_BATCH = 16384

_INFO = plsc.get_sparse_core_info()
_NC = _INFO.num_cores
_NS = _INFO.num_subcores
_NW = _NC * _NS
_BPW = _BATCH // _NW
_IC = 128
_CH = _BPW // _IC


@functools.partial(
    pl.kernel,
    out_type=jax.ShapeDtypeStruct((_NW, _CH, _IC, _ROW), jnp.float32),
    mesh=plsc.VectorSubcoreMesh(core_axis_name="c", subcore_axis_name="s"),
    compiler_params=pltpu.CompilerParams(needs_layout_passes=False),
    scratch_types=[
        pltpu.VMEM((_CH, _IC), jnp.int32),
        pltpu.VMEM((_CH, _IC, _ROW), jnp.float32),
        [pltpu.SemaphoreType.DMA] * _CH,
        pltpu.SemaphoreType.DMA,
    ],
)
def _gather_rows(table_hbm, idx_hbm, out_hbm, idx_v, rows_v, gsems, osem):
    wid = lax.axis_index("s") * _NC + lax.axis_index("c")
    pltpu.sync_copy(idx_hbm.at[wid], idx_v)
    gathers = [
        pltpu.async_copy(table_hbm.at[idx_v.at[j]], rows_v.at[j], gsems[j])
        for j in range(_CH)
    ]
    outs = []
    for j in range(_CH):
        gathers[j].wait()
        outs.append(pltpu.async_copy(rows_v.at[j], out_hbm.at[wid].at[j], osem))
    for o in outs:
        o.wait()


def kernel(boxes, ids):
    num_models, num_boxes, two, dim = boxes.shape
    table = boxes.reshape(num_boxes, two * dim)
    idx = ids.astype(jnp.int32).reshape(_NW, _CH, _IC)
    out = _gather_rows(table, idx)
    return out.reshape(num_models, _BATCH, two, dim)


# speedup vs baseline: 1.3962x; 1.3962x over previous
import functools

import jax
import jax.numpy as jnp
from jax import lax
from jax.experimental import pallas as pl
from jax.experimental.pallas import tpu as pltpu
from jax.experimental.pallas import tpu_sc as plsc

_NUM_BOXES = 100000
_DIM = 64
_NROW = 2 * _DIM
_BATCH = 16384
_HALF = _BATCH // 2

_INFO = plsc.get_sparse_core_info()
_NC = _INFO.num_cores
_NS = _INFO.num_subcores
_NW = _NC * _NS
_R_PER_W = _NROW // _NW
_L = 16
_UNROLL = 8


@functools.partial(
    pl.kernel,
    out_type=jax.ShapeDtypeStruct((_NROW, _BATCH), jnp.float32),
    mesh=plsc.VectorSubcoreMesh(core_axis_name="c", subcore_axis_name="s"),
    compiler_params=pltpu.CompilerParams(needs_layout_passes=False),
    scratch_types=[
        pltpu.VMEM((_BATCH,), jnp.int32),
        pltpu.VMEM((_NUM_BOXES,), jnp.float32),
        pltpu.VMEM((_HALF,), jnp.float32),
    ],
)
def _gather_rows(table_hbm, ids_hbm, out_hbm, ids_v, row_v, out_v):
    wid = lax.axis_index("s") * _NC + lax.axis_index("c")
    pltpu.sync_copy(ids_hbm, ids_v)

    for r in range(_R_PER_W):
        row = wid * _R_PER_W + r
        pltpu.sync_copy(table_hbm.at[row], row_v)

        for h in range(2):
            base = h * _HALF

            def body(i, carry):
                off = base + i * _L * _UNROLL
                for u in range(_UNROLL):
                    idx = ids_v[pl.ds(off + u * _L, _L)]
                    out_v[pl.ds(off - base + u * _L, _L)] = plsc.load_gather(
                        row_v, [idx]
                    )
                return carry

            lax.fori_loop(0, _HALF // (_L * _UNROLL), body, 0)
            pltpu.sync_copy(out_v, out_hbm.at[row, pl.ds(base, _HALF)])


def kernel(boxes, ids):
    num_models, num_boxes, two, dim = boxes.shape
    tableT = jnp.transpose(boxes, (0, 2, 3, 1)).reshape(two * dim, num_boxes)
    outT = _gather_rows(tableT, ids.astype(jnp.int32))
    return jnp.transpose(outT.reshape(num_models, two, dim, _BATCH), (0, 3, 1, 2))


# speedup vs baseline: 1.4313x; 1.0252x over previous
import functools

import jax
import jax.numpy as jnp
from jax import lax
from jax.experimental import pallas as pl
from jax.experimental.pallas import tpu as pltpu
from jax.experimental.pallas import tpu_sc as plsc

_NUM_BOXES = 100000
_DIM = 64
_NROW = 2 * _DIM
_BATCH = 16384
_QT = _BATCH // 4

_INFO = plsc.get_sparse_core_info()
_NC = _INFO.num_cores
_NS = _INFO.num_subcores
_NW = _NC * _NS
_R_PER_W = _NROW // _NW
_L = 16
_UNROLL = 8


@functools.partial(
    pl.kernel,
    out_type=jax.ShapeDtypeStruct((_NROW, _BATCH), jnp.float32),
    mesh=plsc.VectorSubcoreMesh(core_axis_name="c", subcore_axis_name="s"),
    compiler_params=pltpu.CompilerParams(needs_layout_passes=False),
    scratch_types=[
        pltpu.VMEM((_BATCH,), jnp.int32),
        pltpu.VMEM((_NUM_BOXES,), jnp.float32),
        pltpu.VMEM((2 * _QT,), jnp.float32),
        pltpu.SemaphoreType.DMA,
    ],
)
def _gather_rows(table_hbm, ids_hbm, out_hbm, ids_v, row_v, out_v, osem):
    wid = lax.axis_index("s") * _NC + lax.axis_index("c")
    pltpu.sync_copy(ids_hbm, ids_v)
    out_pending = []

    for r in range(_R_PER_W):
        row = wid * _R_PER_W + r
        pltpu.sync_copy(table_hbm.at[row], row_v)

        for q in range(4):
            base = q * _QT
            obase = (q % 2) * _QT
            if len(out_pending) >= 2:
                out_pending.pop(0).wait()

            def body(i, carry):
                off = i * _L * _UNROLL
                for u in range(_UNROLL):
                    idx = ids_v[pl.ds(base + off + u * _L, _L)]
                    out_v[pl.ds(obase + off + u * _L, _L)] = plsc.load_gather(
                        row_v, [idx]
                    )
                return carry

            lax.fori_loop(0, _QT // (_L * _UNROLL), body, 0)
            out_pending.append(
                pltpu.async_copy(
                    out_v.at[pl.ds(obase, _QT)],
                    out_hbm.at[row, pl.ds(base, _QT)],
                    osem,
                )
            )

    for d in out_pending:
        d.wait()


def kernel(boxes, ids):
    num_models, num_boxes, two, dim = boxes.shape
    tableT = jnp.transpose(boxes, (0, 2, 3, 1)).reshape(two * dim, num_boxes)
    outT = _gather_rows(tableT, ids.astype(jnp.int32))
    return jnp.transpose(outT.reshape(num_models, two, dim, _BATCH), (0, 3, 1, 2))
